# baseline (speedup 1.0000x reference)
import functools

import jax
import jax.numpy as jnp
from jax import lax
from jax.experimental import pallas as pl
from jax.experimental.pallas import tpu as pltpu
from jax.experimental.pallas import tpu_sc as plsc

_D = 128
_N = 16
_BATCH = 2048
_S = 128
_T = 1280
_S2 = 128

_NC = 2
_NS = 16
_NW = _NC * _NS
_CH = 8


def _layers_from_means(h0, h1f, m1, m2, w0_ref, aw0_ref, w1_ref, aw1_ref,
                       fcw_ref, fcb_ref, s):
    f32 = jnp.float32
    a1 = jnp.maximum(
        jnp.dot(h1f, w0_ref[...], preferred_element_type=f32)
        + jnp.dot(m2, aw0_ref[...], preferred_element_type=f32), 0.0)
    a0 = jnp.maximum(
        jnp.dot(h0, w0_ref[...], preferred_element_type=f32)
        + jnp.dot(m1, aw0_ref[...], preferred_element_type=f32), 0.0)
    ma1 = jnp.mean(a1.reshape(s, _N, _D), axis=1)
    w1 = w1_ref[...]
    aw1 = aw1_ref[...]
    hid = (jnp.dot(a0, w1[:_D], preferred_element_type=f32)
           + jnp.dot(h0, w1[_D:], preferred_element_type=f32)
           + jnp.dot(ma1, aw1[:_D], preferred_element_type=f32)
           + jnp.dot(m1, aw1[_D:], preferred_element_type=f32))
    fcw = fcw_ref[...]
    return (jnp.dot(hid, fcw[:_D], preferred_element_type=f32)
            + jnp.dot(h0, fcw[_D:], preferred_element_type=f32)
            + fcb_ref[0])


def _fused(h0_ref, h1_ref, h2_ref, w0_ref, aw0_ref, w1_ref, aw1_ref,
           fcw_ref, fcb_ref, out_ref):
    m2 = jnp.mean(h2_ref[...], axis=1)
    h1 = h1_ref[...]
    m1 = jnp.mean(h1, axis=1)
    out_ref[...] = _layers_from_means(
        h0_ref[...], h1.reshape(_S * _N, _D), m1, m2,
        w0_ref, aw0_ref, w1_ref, aw1_ref, fcw_ref, fcb_ref, _S)


def _tail(h0_ref, h1_ref, m2_ref, w0_ref, aw0_ref, w1_ref, aw1_ref,
          fcw_ref, fcb_ref, out_ref):
    h1 = h1_ref[...]
    m1 = jnp.mean(h1, axis=1)
    out_ref[...] = _layers_from_means(
        h0_ref[...], h1.reshape(_S2 * _N, _D), m1, m2_ref[...],
        w0_ref, aw0_ref, w1_ref, aw1_ref, fcw_ref, fcb_ref, _S2)


def _sc_mean_body(h2_hbm, m2_hbm, buf, obuf, sem, *, groups_per_worker):
    wid = lax.axis_index("s") * _NC + lax.axis_index("c")
    g0 = wid * groups_per_worker
    n_chunks = groups_per_worker // _CH
    inv = jnp.full((16,), 1.0 / _N, dtype=jnp.float32)

    def chunk_body(k, _):
        gbase = g0 + k * _CH
        pltpu.async_copy(
            h2_hbm.at[pl.ds((_T * _N + gbase) * _N, _CH * _N), :],
            buf, sem).wait()

        def group_body(g, _):
            rbase = g * _N
            for c in range(_D // 16):
                acc = buf[rbase, pl.ds(c * 16, 16)]
                for r in range(1, _N):
                    acc = acc + buf[rbase + r, pl.ds(c * 16, 16)]
                obuf[g, pl.ds(c * 16, 16)] = acc * inv
            return 0

        lax.fori_loop(0, _CH, group_body, 0)
        pltpu.sync_copy(obuf, m2_hbm.at[pl.ds(gbase, _CH), :])
        return 0

    lax.fori_loop(0, n_chunks, chunk_body, 0)


def _sc_mean(h2):
    n_tail_groups = (_BATCH - _T) * _N
    gpw = n_tail_groups // _NW
    mesh = plsc.VectorSubcoreMesh(core_axis_name="c", subcore_axis_name="s",
                                  num_cores=_NC, num_subcores=_NS)
    k = pl.kernel(
        functools.partial(_sc_mean_body, groups_per_worker=gpw),
        out_type=jax.ShapeDtypeStruct((n_tail_groups, _D), jnp.float32),
        mesh=mesh,
        scratch_types=[
            pltpu.VMEM((_CH * _N, _D), jnp.float32),
            pltpu.VMEM((_CH, _D), jnp.float32),
            pltpu.SemaphoreType.DMA,
        ],
    )
    return k(h2)


def _tc_calls(h0, h1, h2, m2_tail, w0, aw0, w1, aw1, fc_w, fc_b):
    h1r = h1.reshape(_BATCH, _N, _D)
    h2r = h2.reshape(_BATCH * _N, _N, _D)
    rep2 = lambda i: (0, 0)
    wspecs = [
        pl.BlockSpec((_D, _D), rep2),
        pl.BlockSpec((_D, _D), rep2),
        pl.BlockSpec((2 * _D, _D), rep2),
        pl.BlockSpec((2 * _D, _D), rep2),
        pl.BlockSpec((2 * _D, 1), rep2),
        pl.BlockSpec(memory_space=pltpu.SMEM),
    ]
    head = pl.pallas_call(
        _fused,
        grid=(_T // _S,),
        in_specs=[
            pl.BlockSpec((_S, _D), lambda i: (i, 0)),
            pl.BlockSpec((_S, _N, _D), lambda i: (i, 0, 0)),
            pl.BlockSpec((_S * _N, _N, _D), lambda i: (i, 0, 0)),
        ] + wspecs,
        out_specs=pl.BlockSpec((_S, 1), lambda i: (i, 0)),
        out_shape=jax.ShapeDtypeStruct((_T, 1), jnp.float32),
        compiler_params=pltpu.CompilerParams(
            dimension_semantics=("arbitrary",),
        ),
    )(h0, h1r, h2r, w0, aw0, w1, aw1, fc_w, fc_b)
    toff = _T // _S2
    tail = pl.pallas_call(
        _tail,
        grid=((_BATCH - _T) // _S2,),
        in_specs=[
            pl.BlockSpec((_S2, _D), lambda i: (i + toff, 0)),
            pl.BlockSpec((_S2, _N, _D), lambda i: (i + toff, 0, 0)),
            pl.BlockSpec((_S2 * _N, _D), lambda i: (i, 0)),
        ] + wspecs,
        out_specs=pl.BlockSpec((_S2, 1), lambda i: (i, 0)),
        out_shape=jax.ShapeDtypeStruct((_BATCH - _T, 1), jnp.float32),
        compiler_params=pltpu.CompilerParams(
            dimension_semantics=("arbitrary",),
        ),
    )(h0, h1r, m2_tail, w0, aw0, w1, aw1, fc_w, fc_b)
    return head, tail


def kernel(node_features_list_0, node_features_list_1, node_features_list_2,
           w0, aw0, w1, aw1, fc_w, fc_b):
    h2 = node_features_list_2
    m2_tail = _sc_mean(h2)
    head, tail = _tc_calls(node_features_list_0, node_features_list_1, h2,
                           m2_tail, w0, aw0, w1, aw1, fc_w, fc_b)
    return jnp.concatenate([head, tail], axis=0)


# speedup vs baseline: 1.5341x; 1.5341x over previous
import functools

import jax
import jax.numpy as jnp
from jax import lax
from jax.experimental import pallas as pl
from jax.experimental.pallas import tpu as pltpu
from jax.experimental.pallas import tpu_sc as plsc

_D = 128
_N = 16
_BATCH = 2048
_S = 128
_T = 1792
_S2 = 128
_NC = 2
_NS = 16
_NW = _NC * _NS
_CH = 16


def _layers_from_means(h0, h1f, m1, m2, w0_ref, aw0_ref, w1_ref, aw1_ref,
                       fcw_ref, fcb_ref, s):
    f32 = jnp.float32
    a1 = jnp.maximum(
        jnp.dot(h1f, w0_ref[...], preferred_element_type=f32)
        + jnp.dot(m2, aw0_ref[...], preferred_element_type=f32), 0.0)
    a0 = jnp.maximum(
        jnp.dot(h0, w0_ref[...], preferred_element_type=f32)
        + jnp.dot(m1, aw0_ref[...], preferred_element_type=f32), 0.0)
    ma1 = jnp.mean(a1.reshape(s, _N, _D), axis=1)
    w1 = w1_ref[...]
    aw1 = aw1_ref[...]
    hid = (jnp.dot(a0, w1[:_D], preferred_element_type=f32)
           + jnp.dot(h0, w1[_D:], preferred_element_type=f32)
           + jnp.dot(ma1, aw1[:_D], preferred_element_type=f32)
           + jnp.dot(m1, aw1[_D:], preferred_element_type=f32))
    fcw = fcw_ref[...]
    return (jnp.dot(hid, fcw[:_D], preferred_element_type=f32)
            + jnp.dot(h0, fcw[_D:], preferred_element_type=f32)
            + fcb_ref[0])


def _fused(h0_ref, h1_ref, h2_ref, w0_ref, aw0_ref, w1_ref, aw1_ref,
           fcw_ref, fcb_ref, out_ref):
    m2 = jnp.mean(h2_ref[...], axis=1)
    h1 = h1_ref[...]
    m1 = jnp.mean(h1, axis=1)
    out_ref[...] = _layers_from_means(
        h0_ref[...], h1.reshape(_S * _N, _D), m1, m2,
        w0_ref, aw0_ref, w1_ref, aw1_ref, fcw_ref, fcb_ref, _S)


def _tail(h0_ref, h1_ref, m2_ref, w0_ref, aw0_ref, w1_ref, aw1_ref,
          fcw_ref, fcb_ref, out_ref):
    h1 = h1_ref[...]
    m1 = jnp.mean(h1, axis=1)
    out_ref[...] = _layers_from_means(
        h0_ref[...], h1.reshape(_S2 * _N, _D), m1, m2_ref[...],
        w0_ref, aw0_ref, w1_ref, aw1_ref, fcw_ref, fcb_ref, _S2)


def _sc_mean_body(h2_hbm, m2_hbm, buf0, buf1, obuf0, obuf1,
                  sem0, sem1, osem0, osem1, *, groups_per_worker):
    wid = lax.axis_index("s") * _NC + lax.axis_index("c")
    g0 = wid * groups_per_worker
    n_chunks = groups_per_worker // _CH
    inv = jnp.full((16,), 1.0 / _N, dtype=jnp.float32)
    bufs, obufs = (buf0, buf1), (obuf0, obuf1)
    sems, osems = (sem0, sem1), (osem0, osem1)

    def in_slice(k):
        return h2_hbm.at[pl.ds((_T * _N + g0 + k * _CH) * _N, _CH * _N), :]

    def out_slice(k):
        return m2_hbm.at[pl.ds(g0 + k * _CH, _CH), :]

    for b in (0, 1):
        pltpu.async_copy(in_slice(jnp.int32(b)), bufs[b], sems[b])

    def pair_body(k2, _):
        for b in (0, 1):
            k = 2 * k2 + b
            pltpu.make_async_copy(in_slice(k), bufs[b], sems[b]).wait()

            @pl.when(k2 > 0)
            def _():
                pltpu.make_async_copy(obufs[b], out_slice(k - 2),
                                      osems[b]).wait()

            def group_body(g, _):
                rbase = g * _N
                for c in range(_D // 16):
                    acc = bufs[b][rbase, pl.ds(c * 16, 16)]
                    for r in range(1, _N):
                        acc = acc + bufs[b][rbase + r, pl.ds(c * 16, 16)]
                    obufs[b][g, pl.ds(c * 16, 16)] = acc * inv
                return 0

            lax.fori_loop(0, _CH, group_body, 0, unroll=2)
            pltpu.async_copy(obufs[b], out_slice(k), osems[b])
            knext = jnp.minimum(k + 2, n_chunks - 1)
            pltpu.async_copy(in_slice(knext), bufs[b], sems[b])
        return 0

    lax.fori_loop(0, n_chunks // 2, pair_body, 0)
    for b in (0, 1):
        pltpu.make_async_copy(in_slice(jnp.int32(0)), bufs[b], sems[b]).wait()
        pltpu.make_async_copy(obufs[b], out_slice(jnp.int32(0)),
                              osems[b]).wait()


def _sc_mean(h2):
    n_tail_groups = (_BATCH - _T) * _N
    gpw = n_tail_groups // _NW
    mesh = plsc.VectorSubcoreMesh(core_axis_name="c", subcore_axis_name="s",
                                  num_cores=_NC, num_subcores=_NS)
    k = pl.kernel(
        functools.partial(_sc_mean_body, groups_per_worker=gpw),
        out_type=jax.ShapeDtypeStruct((n_tail_groups, _D), jnp.float32),
        mesh=mesh,
        scratch_types=[
            pltpu.VMEM((_CH * _N, _D), jnp.float32),
            pltpu.VMEM((_CH * _N, _D), jnp.float32),
            pltpu.VMEM((_CH, _D), jnp.float32),
            pltpu.VMEM((_CH, _D), jnp.float32),
            pltpu.SemaphoreType.DMA,
            pltpu.SemaphoreType.DMA,
            pltpu.SemaphoreType.DMA,
            pltpu.SemaphoreType.DMA,
        ],
    )
    return k(h2)


def _tc_calls(h0, h1, h2, m2_tail, w0, aw0, w1, aw1, fc_w, fc_b):
    h1r = h1.reshape(_BATCH, _N, _D)
    h2r = h2.reshape(_BATCH * _N, _N, _D)
    rep2 = lambda i: (0, 0)
    wspecs = [
        pl.BlockSpec((_D, _D), rep2),
        pl.BlockSpec((_D, _D), rep2),
        pl.BlockSpec((2 * _D, _D), rep2),
        pl.BlockSpec((2 * _D, _D), rep2),
        pl.BlockSpec((2 * _D, 1), rep2),
        pl.BlockSpec(memory_space=pltpu.SMEM),
    ]
    head = pl.pallas_call(
        _fused,
        grid=(_T // _S,),
        in_specs=[
            pl.BlockSpec((_S, _D), lambda i: (i, 0)),
            pl.BlockSpec((_S, _N, _D), lambda i: (i, 0, 0)),
            pl.BlockSpec((_S * _N, _N, _D), lambda i: (i, 0, 0)),
        ] + wspecs,
        out_specs=pl.BlockSpec((_S, 1), lambda i: (i, 0)),
        out_shape=jax.ShapeDtypeStruct((_T, 1), jnp.float32),
        compiler_params=pltpu.CompilerParams(
            dimension_semantics=("arbitrary",),
        ),
    )(h0, h1r, h2r, w0, aw0, w1, aw1, fc_w, fc_b)
    toff = _T // _S2
    tail = pl.pallas_call(
        _tail,
        grid=((_BATCH - _T) // _S2,),
        in_specs=[
            pl.BlockSpec((_S2, _D), lambda i: (i + toff, 0)),
            pl.BlockSpec((_S2, _N, _D), lambda i: (i + toff, 0, 0)),
            pl.BlockSpec((_S2 * _N, _D), lambda i: (i, 0)),
        ] + wspecs,
        out_specs=pl.BlockSpec((_S2, 1), lambda i: (i, 0)),
        out_shape=jax.ShapeDtypeStruct((_BATCH - _T, 1), jnp.float32),
        compiler_params=pltpu.CompilerParams(
            dimension_semantics=("arbitrary",),
        ),
    )(h0, h1r, m2_tail, w0, aw0, w1, aw1, fc_w, fc_b)
    return head, tail


def kernel(node_features_list_0, node_features_list_1, node_features_list_2,
           w0, aw0, w1, aw1, fc_w, fc_b):
    h2 = node_features_list_2
    m2_tail = _sc_mean(h2)
    head, tail = _tc_calls(node_features_list_0, node_features_list_1, h2,
                           m2_tail, w0, aw0, w1, aw1, fc_w, fc_b)
    return jnp.concatenate([head, tail], axis=0)


# speedup vs baseline: 1.9385x; 1.2636x over previous
import jax
import jax.numpy as jnp
from jax.experimental import pallas as pl
from jax.experimental.pallas import tpu as pltpu

_D = 128
_N = 16
_BATCH = 2048
_S = 128


def _fused(h0_ref, h1_ref, h2_ref, w0_ref, aw0_ref, w1_ref, aw1_ref,
           fcw_ref, fcb_ref, out_ref):
    f32 = jnp.float32
    m2 = jnp.mean(h2_ref[...], axis=1)
    h1 = h1_ref[...]
    h1f = h1.reshape(_S * _N, _D)
    a1 = jnp.maximum(
        jnp.dot(h1f, w0_ref[...], preferred_element_type=f32)
        + jnp.dot(m2, aw0_ref[...], preferred_element_type=f32), 0.0)
    m1 = jnp.mean(h1, axis=1)
    h0 = h0_ref[...]
    a0 = jnp.maximum(
        jnp.dot(h0, w0_ref[...], preferred_element_type=f32)
        + jnp.dot(m1, aw0_ref[...], preferred_element_type=f32), 0.0)
    ma1 = jnp.mean(a1.reshape(_S, _N, _D), axis=1)
    w1 = w1_ref[...]
    aw1 = aw1_ref[...]
    hid = (jnp.dot(a0, w1[:_D], preferred_element_type=f32)
           + jnp.dot(h0, w1[_D:], preferred_element_type=f32)
           + jnp.dot(ma1, aw1[:_D], preferred_element_type=f32)
           + jnp.dot(m1, aw1[_D:], preferred_element_type=f32))
    fcw = fcw_ref[...]
    out_ref[...] = (jnp.dot(hid, fcw[:_D], preferred_element_type=f32)
                    + jnp.dot(h0, fcw[_D:], preferred_element_type=f32)
                    + fcb_ref[0])


def kernel(node_features_list_0, node_features_list_1, node_features_list_2,
           w0, aw0, w1, aw1, fc_w, fc_b):
    h0 = node_features_list_0
    h1 = node_features_list_1.reshape(_BATCH, _N, _D)
    h2 = node_features_list_2.reshape(_BATCH * _N, _N, _D)
    grid = (_BATCH // _S,)
    rep2 = lambda i: (0, 0)
    return pl.pallas_call(
        _fused,
        grid=grid,
        in_specs=[
            pl.BlockSpec((_S, _D), lambda i: (i, 0)),
            pl.BlockSpec((_S, _N, _D), lambda i: (i, 0, 0)),
            pl.BlockSpec((_S * _N, _N, _D), lambda i: (i, 0, 0)),
            pl.BlockSpec((_D, _D), rep2),
            pl.BlockSpec((_D, _D), rep2),
            pl.BlockSpec((2 * _D, _D), rep2),
            pl.BlockSpec((2 * _D, _D), rep2),
            pl.BlockSpec((2 * _D, 1), rep2),
            pl.BlockSpec(memory_space=pltpu.SMEM),
        ],
        out_specs=pl.BlockSpec((_S, 1), lambda i: (i, 0)),
        out_shape=jax.ShapeDtypeStruct((_BATCH, 1), jnp.float32),
        compiler_params=pltpu.CompilerParams(
            dimension_semantics=("arbitrary",),
        ),
    )(h0, h1, h2, w0, aw0, w1, aw1, fc_w, fc_b)
